# Initial kernel scaffold; baseline (speedup 1.0000x reference)
import jax
import jax.numpy as jnp
from jax.experimental import pallas as pl


def kernel(token_ids, weight):
    raise NotImplementedError("write your pallas kernel here")


# speedup vs baseline: 1.7962x; 1.7962x over previous
import functools

import jax
import jax.numpy as jnp
from jax import lax
from jax.experimental import pallas as pl
from jax.experimental.pallas import tpu as pltpu
from jax.experimental.pallas import tpu_sc as plsc

_IDXW = 128
_K = 4


@functools.lru_cache(maxsize=None)
def _build(V, D, B):
  info = plsc.get_sparse_core_info()
  NC, NS = info.num_cores, info.num_subcores
  NW = NC * NS
  rows = B // _IDXW
  rows_per_w = rows // NW
  C = _K * _IDXW
  n_chunks = rows_per_w // _K

  mesh = plsc.VectorSubcoreMesh(core_axis_name="c", subcore_axis_name="s")

  @functools.partial(
      pl.kernel,
      mesh=mesh,
      out_type=jax.ShapeDtypeStruct((B, D), jnp.float32),
      scratch_types=[
          pltpu.VMEM((_K, _IDXW), jnp.int32),
          pltpu.VMEM((C, D), jnp.float32),
          pltpu.SemaphoreType.DMA,
      ],
      compiler_params=pltpu.CompilerParams(use_tc_tiling_on_sc=False),
  )
  def gather_kernel(table_hbm, idx_hbm, out_hbm, idx_v, rows_v, sem):
    wid = lax.axis_index("s") * NC + lax.axis_index("c")
    row0 = wid * rows_per_w
    base = row0 * _IDXW

    def body(i, carry):
      pltpu.sync_copy(idx_hbm.at[pl.ds(row0 + i * _K, _K), :], idx_v)
      copies = [
          pltpu.async_copy(
              table_hbm.at[idx_v.at[j]],
              rows_v.at[pl.ds(j * _IDXW, _IDXW), :],
              sem,
          )
          for j in range(_K)
      ]
      for cp in copies:
        cp.wait()
      pltpu.sync_copy(rows_v, out_hbm.at[pl.ds(base + i * C, C), :])
      return carry

    lax.fori_loop(0, n_chunks, body, 0)

  return gather_kernel


def kernel(token_ids, weight):
  S, T = token_ids.shape
  V, D = weight.shape
  B = S * T
  idx2d = token_ids.reshape(B // _IDXW, _IDXW)
  out = _build(V, D, B)(weight, idx2d)
  return out.reshape(S, T, D)


# speedup vs baseline: 1.8537x; 1.0320x over previous
import functools

import jax
import jax.numpy as jnp
from jax import lax
from jax.experimental import pallas as pl
from jax.experimental.pallas import tpu as pltpu
from jax.experimental.pallas import tpu_sc as plsc

_IDXW = 128
_K = 4


@functools.lru_cache(maxsize=None)
def _build(V, D, B):
  info = plsc.get_sparse_core_info()
  NC, NS = info.num_cores, info.num_subcores
  NW = NC * NS
  rows = B // _IDXW
  rows_per_w = rows // NW
  C = _K * _IDXW
  n_chunks = rows_per_w // _K

  mesh = plsc.VectorSubcoreMesh(core_axis_name="c", subcore_axis_name="s")

  @functools.partial(
      pl.kernel,
      mesh=mesh,
      out_type=jax.ShapeDtypeStruct((B, D), jnp.float32),
      scratch_types=[
          pltpu.VMEM((2, _K, _IDXW), jnp.int32),
          pltpu.VMEM((2, C, D), jnp.float32),
          pltpu.SemaphoreType.DMA((2,)),
      ],
      compiler_params=pltpu.CompilerParams(use_tc_tiling_on_sc=False),
  )
  def gather_kernel(table_hbm, idx_hbm, out_hbm, idx_v, rows_v, sems):
    wid = lax.axis_index("s") * NC + lax.axis_index("c")
    row0 = wid * rows_per_w
    base = row0 * _IDXW

    def fire(i, b):
      pltpu.sync_copy(idx_hbm.at[pl.ds(row0 + i * _K, _K), :], idx_v.at[b])
      for j in range(_K):
        pltpu.async_copy(
            table_hbm.at[idx_v.at[b, j]],
            rows_v.at[b, pl.ds(j * _IDXW, _IDXW), :],
            sems.at[b],
        )

    def wait_store(i, b):
      for j in range(_K):
        pltpu.make_async_copy(
            table_hbm.at[idx_v.at[b, j]],
            rows_v.at[b, pl.ds(j * _IDXW, _IDXW), :],
            sems.at[b],
        ).wait()
      pltpu.sync_copy(rows_v.at[b], out_hbm.at[pl.ds(base + i * C, C), :])

    fire(0, 0)
    fire(1, 1)

    @pl.loop(0, n_chunks - 2, step=2)
    def _steady(g):
      for b in range(2):
        wait_store(g + b, b)
        fire(g + b + 2, b)

    for b in range(2):
      wait_store(n_chunks - 2 + b, b)

  return gather_kernel


def kernel(token_ids, weight):
  S, T = token_ids.shape
  V, D = weight.shape
  B = S * T
  idx2d = token_ids.reshape(B // _IDXW, _IDXW)
  out = _build(V, D, B)(weight, idx2d)
  return out.reshape(S, T, D)


# speedup vs baseline: 2.3514x; 1.2684x over previous
import functools

import jax
import jax.numpy as jnp
from jax import lax
from jax.experimental import pallas as pl
from jax.experimental.pallas import tpu as pltpu
from jax.experimental.pallas import tpu_sc as plsc

_L = 128
_PITCH = 129


@functools.lru_cache(maxsize=None)
def _build(V, D, S, T):
  info = plsc.get_sparse_core_info()
  NC, NS = info.num_cores, info.num_subcores
  NW = NC * NS
  NT = S // _L
  n_blocks = T * NT
  bpw = n_blocks // NW
  G = D // 8

  mesh = plsc.VectorSubcoreMesh(core_axis_name="c", subcore_axis_name="s")

  @functools.partial(
      pl.kernel,
      mesh=mesh,
      out_type=jax.ShapeDtypeStruct((T, G, NT, 8, _L), jnp.float32),
      scratch_types=[
          pltpu.VMEM((bpw, _L), jnp.int32),
          pltpu.VMEM((2, _L, D), jnp.float32),
          pltpu.VMEM((2, D, _PITCH), jnp.float32),
          pltpu.SemaphoreType.DMA((2,)),
          pltpu.SemaphoreType.DMA((2,)),
      ],
      compiler_params=pltpu.CompilerParams(use_tc_tiling_on_sc=False,
                                           needs_layout_passes=False),
  )
  def gather_kernel(table_hbm, idx_hbm, out_hbm, idx_v, emb_v, outt_v,
                    gsems, ssems):
    wid = lax.axis_index("s") * NC + lax.axis_index("c")
    n0 = wid * bpw
    pltpu.sync_copy(idx_hbm.at[pl.ds(n0, bpw), :], idx_v)

    iota = lax.iota(jnp.int32, 16)
    rowvs = [iota + d0 for d0 in range(0, D, 16)]

    def gather(j, b, make):
      return make(table_hbm.at[idx_v.at[j]], emb_v.at[b], gsems.at[b])

    def stores(j, b, make):
      t = (n0 + j) // NT
      c = (n0 + j) % NT
      return [
          make(outt_v.at[b, pl.ds(8 * g, 8), pl.ds(0, _L)],
               out_hbm.at[t, g, c], ssems.at[b])
          for g in range(G)
      ]

    def transpose(b):
      @pl.loop(0, _L, unroll=2)
      def _t(l):
        colv = jnp.full((16,), 0, jnp.int32) + l
        for k, d0 in enumerate(range(0, D, 16)):
          v = emb_v[b, l, pl.ds(d0, 16)]
          plsc.store_scatter(outt_v.at[b], [rowvs[k], colv], v)

    gather(0, 0, pltpu.async_copy)
    gather(1, 1, pltpu.async_copy)

    @pl.loop(0, bpw, step=2)
    def _steady(jj):
      for b in range(2):
        j = jj + b
        gather(j, b, pltpu.make_async_copy).wait()

        @pl.when(jj >= 2)
        def _drain():
          for cp in stores(j - 2, b, pltpu.make_async_copy):
            cp.wait()

        transpose(b)
        stores(j, b, pltpu.async_copy)

        @pl.when(jj + 2 + b < bpw)
        def _prefetch():
          gather(j + 2, b, pltpu.async_copy)

    for b in range(2):
      for cp in stores(bpw - 2 + b, b, pltpu.make_async_copy):
        cp.wait()

  return gather_kernel


def kernel(token_ids, weight):
  S, T = token_ids.shape
  V, D = weight.shape
  idxf = token_ids.T.reshape(T * (S // _L), _L)
  out5 = _build(V, D, S, T)(weight, idxf)
  return out5.transpose(2, 4, 0, 1, 3).reshape(S, T, D)


# speedup vs baseline: 2.9824x; 1.2684x over previous
import functools

import jax
import jax.numpy as jnp
from jax import lax
from jax.experimental import pallas as pl
from jax.experimental.pallas import tpu as pltpu
from jax.experimental.pallas import tpu_sc as plsc

_L = 128
_PITCH = 129


@functools.lru_cache(maxsize=None)
def _build(V, D, S, T):
  info = plsc.get_sparse_core_info()
  NC, NS = info.num_cores, info.num_subcores
  NW = NC * NS
  NT = S // _L
  n_blocks = T * NT
  bpw = n_blocks // NW
  G = D // 8

  mesh = plsc.VectorSubcoreMesh(core_axis_name="c", subcore_axis_name="s")

  @functools.partial(
      pl.kernel,
      mesh=mesh,
      out_type=jax.ShapeDtypeStruct((T, G, NT, 8, _L), jnp.float32),
      scratch_types=[
          pltpu.VMEM((bpw, _L), jnp.int32),
          pltpu.VMEM((2, _L, D), jnp.float32),
          pltpu.VMEM((2, D, _PITCH), jnp.float32),
          pltpu.SemaphoreType.DMA((2,)),
          pltpu.SemaphoreType.DMA((2,)),
      ],
      compiler_params=pltpu.CompilerParams(use_tc_tiling_on_sc=False,
                                           needs_layout_passes=False),
  )
  def gather_kernel(table_hbm, idx_hbm, out_hbm, idx_v, emb_v, outt_v,
                    gsems, ssems):
    wid = lax.axis_index("s") * NC + lax.axis_index("c")
    n0 = wid * bpw
    pltpu.sync_copy(idx_hbm.at[pl.ds(n0, bpw), :], idx_v)

    iota = lax.iota(jnp.int32, 16)
    rowvs = [iota + d0 for d0 in range(0, D, 16)]

    def gather(j, b, make):
      return make(table_hbm.at[idx_v.at[j]], emb_v.at[b], gsems.at[b])

    def stores(j, b, make):
      t = (n0 + j) // NT
      c = (n0 + j) % NT
      return [
          make(outt_v.at[b, pl.ds(8 * g, 8), pl.ds(0, _L)],
               out_hbm.at[t, g, c], ssems.at[b])
          for g in range(G)
      ]

    def transpose(b):
      @plsc.parallel_loop(0, _L, unroll=4)
      def _t(l):
        colv = jnp.full((16,), 0, jnp.int32) + l
        for k, d0 in enumerate(range(0, D, 16)):
          v = emb_v[b, l, pl.ds(d0, 16)]
          plsc.store_scatter(outt_v.at[b], [rowvs[k], colv], v)

    gather(0, 0, pltpu.async_copy)
    gather(1, 1, pltpu.async_copy)

    @pl.loop(0, bpw, step=2)
    def _steady(jj):
      for b in range(2):
        j = jj + b
        gather(j, b, pltpu.make_async_copy).wait()

        @pl.when(jj >= 2)
        def _drain():
          for cp in stores(j - 2, b, pltpu.make_async_copy):
            cp.wait()

        transpose(b)
        stores(j, b, pltpu.async_copy)

        @pl.when(jj + 2 + b < bpw)
        def _prefetch():
          gather(j + 2, b, pltpu.async_copy)

    for b in range(2):
      for cp in stores(bpw - 2 + b, b, pltpu.make_async_copy):
        cp.wait()

  return gather_kernel


def kernel(token_ids, weight):
  S, T = token_ids.shape
  V, D = weight.shape
  idxf = token_ids.T.reshape(T * (S // _L), _L)
  out5 = _build(V, D, S, T)(weight, idxf)
  return out5.transpose(2, 4, 0, 1, 3).reshape(S, T, D)
